# baseline (speedup 1.0000x reference)
import functools

import jax
import jax.numpy as jnp
from jax import lax
from jax.experimental import pallas as pl
from jax.experimental.pallas import tpu as pltpu
from jax.experimental.pallas import tpu_sc as plsc

D = 64
NC, NS, L = 2, 16, 16
NW = NC * NS
B = 4096
H = 200
G = 256
NBP = B // G
NGRP = H * NBP
PER_W = NGRP // NW
NV = G // L
RN = 3

_mesh = plsc.VectorSubcoreMesh(core_axis_name="c", subcore_axis_name="s")


@functools.partial(
    pl.kernel,
    out_type=jax.ShapeDtypeStruct((H, D // 8, B // 128, 8, 128), jnp.float32),
    mesh=_mesh,
    scratch_types=[
        pltpu.VMEM((RN, G), jnp.int32),
        pltpu.VMEM((RN, G), jnp.int32),
        pltpu.VMEM((RN, G), jnp.int32),
        pltpu.VMEM((G,), jnp.int32),
        pltpu.VMEM((RN, G, 128), jnp.float32),
        pltpu.VMEM((2, 2, 2, 8, 128), jnp.float32),
        pltpu.SemaphoreType.DMA((RN,)),
        pltpu.SemaphoreType.DMA((RN,)),
        pltpu.SemaphoreType.DMA((2,)),
    ],
    compiler_params=pltpu.CompilerParams(
        use_tc_tiling_on_sc=False, needs_layout_passes=False),
)
def _emb_lookup(xf_hbm, wv_hbm, out_hbm, idx_v, idxf_v, base_v, rowi_v,
                rows_v, outt_v, s_idx, s_g, s_o):
    wid = lax.axis_index("s") * NC + lax.axis_index("c")
    j0 = wid * PER_W

    for k in range(NV):
        rowi_v[pl.ds(k * L, L)] = lax.iota(jnp.int32, L) + (k * L)

    def q_of(j):
        g2 = j0 + j
        h = lax.shift_right_logical(g2, 4)
        btp = lax.bitwise_and(g2, NBP - 1)
        return h, btp, h * B + btp * G

    def start_idx(p, j):
        _, _, q0 = q_of(j)
        pltpu.async_copy(xf_hbm.at[pl.ds(q0, G)], idx_v.at[p], s_idx.at[p])

    def wait_idx(p):
        pltpu.make_async_copy(
            xf_hbm.at[pl.ds(0, G)], idx_v.at[p], s_idx.at[p]).wait()

    def fuse(p):
        for k in range(NV):
            v = idx_v[p, pl.ds(k * L, L)]
            idxf_v[p, pl.ds(k * L, L)] = lax.shift_right_logical(v, 1)
            base_v[p, pl.ds(k * L, L)] = lax.shift_left(
                lax.bitwise_and(v, 1), 6)

    def start_gather(p):
        pltpu.async_copy(wv_hbm.at[idxf_v.at[p]], rows_v.at[p], s_g.at[p])

    def wait_gather(p):
        pltpu.make_async_copy(
            wv_hbm.at[idxf_v.at[p]], rows_v.at[p], s_g.at[p]).wait()

    def transpose_q(rp, ph, q):
        @pl.loop(0, 2)
        def _dtl(dtl):
            d8 = (q * 2 + dtl) * 8

            @pl.loop(0, NV, unroll=2)
            def _k(k):
                off = k * L
                row = rowi_v[pl.ds(off, L)]
                cb = base_v[rp, pl.ds(off, L)]
                btl = lax.shift_right_logical(k, 3)
                k2o = lax.bitwise_and(k, 7) * L
                for dp in range(8):
                    outt_v[ph, dtl, btl, dp, pl.ds(k2o, L)] = (
                        plsc.load_gather(
                            rows_v.at[rp], [row, cb + (d8 + dp)]))

    def start_write(ph, j, q):
        h, btp, _ = q_of(j)
        pltpu.async_copy(
            outt_v.at[ph],
            out_hbm.at[h, pl.ds(q * 2, 2), pl.ds(btp * 2, 2)], s_o.at[ph])

    def wait_write(ph, j, q):
        h, btp, _ = q_of(j)
        pltpu.make_async_copy(
            outt_v.at[ph],
            out_hbm.at[h, pl.ds(q * 2, 2), pl.ds(btp * 2, 2)], s_o.at[ph]
        ).wait()

    start_idx(0, 0)
    start_idx(1, 1)
    wait_idx(0)
    fuse(0)
    start_gather(0)
    start_idx(2, 2)
    wait_idx(1)
    fuse(1)
    start_gather(1)
    wait_idx(2)
    fuse(2)
    start_gather(2)
    start_idx(0, 3)
    wait_gather(0)
    for q in range(4):
        ph = q % 2
        if q >= 2:
            wait_write(ph, 0, q - 2)
        transpose_q(0, ph, q)
        start_write(ph, 0, q)

    @pl.loop(0, (PER_W - 1) // RN)
    def _grp(t):
        for rr in range(RN):
            j = 1 + t * RN + rr
            rj = (1 + rr) % RN
            r2 = (rr) % RN

            @pl.when(j <= PER_W - 3)
            def _():
                wait_idx(r2)
                fuse(r2)
                start_gather(r2)

            @pl.when(j <= PER_W - 4)
            def _():
                start_idx(rj, j + 3)

            wait_gather(rj)
            for q in range(4):
                ph = q % 2
                if q < 2:
                    wait_write(ph, j - 1, q + 2)
                else:
                    wait_write(ph, j, q - 2)
                transpose_q(rj, ph, q)
                start_write(ph, j, q)

    wait_write(0, PER_W - 1, 2)
    wait_write(1, PER_W - 1, 3)


def kernel(x, weight):
    xf = x.T.reshape(B * H)
    wv = weight.reshape(500000, 128)
    out5 = _emb_lookup(xf, wv)
    return out5.transpose((2, 4, 0, 1, 3)).reshape(B, H, D)


# speedup vs baseline: 1.2805x; 1.2805x over previous
import functools

import jax
import jax.numpy as jnp
from jax import lax
from jax.experimental import pallas as pl
from jax.experimental.pallas import tpu as pltpu
from jax.experimental.pallas import tpu_sc as plsc

D = 64
NC, NS, L = 2, 16, 16
NW = NC * NS
B = 4096
H = 200
G = 256
NBP = B // G
NGRP = H * NBP
PER_W = NGRP // NW
NV = G // L
RN = 3
W = 257

_mesh = plsc.VectorSubcoreMesh(core_axis_name="c", subcore_axis_name="s")


@functools.partial(
    pl.kernel,
    out_type=jax.ShapeDtypeStruct((H, D // 8, B // 128, 8, 128), jnp.float32),
    mesh=_mesh,
    scratch_types=[
        pltpu.VMEM((RN, G), jnp.int32),
        pltpu.VMEM((RN, G), jnp.int32),
        pltpu.VMEM((RN, G), jnp.int32),
        pltpu.VMEM((RN, G, 128), jnp.float32),
        pltpu.VMEM((4, L, W), jnp.float32),
        pltpu.SemaphoreType.DMA((RN,)),
        pltpu.SemaphoreType.DMA((RN,)),
        pltpu.SemaphoreType.DMA,
    ],
    compiler_params=pltpu.CompilerParams(
        use_tc_tiling_on_sc=False, needs_layout_passes=False),
)
def _emb_lookup(xf_hbm, wv_hbm, out_hbm, idx_v, idxf_v, par_v,
                rows_v, outq_v, s_idx, s_g, s_o):
    wid = lax.axis_index("s") * NC + lax.axis_index("c")
    j0 = wid * PER_W

    def q_of(j):
        g2 = j0 + j
        h = lax.shift_right_logical(g2, 4)
        btp = lax.bitwise_and(g2, NBP - 1)
        return h, btp, h * B + btp * G

    def start_idx(p, j):
        _, _, q0 = q_of(j)
        pltpu.async_copy(xf_hbm.at[pl.ds(q0, G)], idx_v.at[p], s_idx.at[p])

    def wait_idx(p):
        pltpu.make_async_copy(
            xf_hbm.at[pl.ds(0, G)], idx_v.at[p], s_idx.at[p]).wait()

    def fuse(p):
        for k in range(NV):
            v = idx_v[p, pl.ds(k * L, L)]
            idxf_v[p, pl.ds(k * L, L)] = lax.shift_right_logical(v, 1)
            par_v[p, pl.ds(k * L, L)] = lax.bitwise_and(v, 1)

    def start_gather(p):
        pltpu.async_copy(wv_hbm.at[idxf_v.at[p]], rows_v.at[p], s_g.at[p])

    def wait_gather(p):
        pltpu.make_async_copy(
            wv_hbm.at[idxf_v.at[p]], rows_v.at[p], s_g.at[p]).wait()

    def scatter_group(rj):
        rowi = lax.iota(jnp.int32, L)

        @pl.loop(0, G, unroll=2)
        def _j(j):
            col = jnp.full((L,), 0, jnp.int32) + j
            par = plsc.load_gather(par_v.at[rj], [col])
            m0 = par == 0
            m1 = par == 1
            for t in range(8):
                q = t if t < 4 else t - 4
                m = m0 if t < 4 else m1
                val = rows_v[rj, j, pl.ds(t * L, L)]
                plsc.store_scatter(outq_v.at[q], [rowi, col], val, mask=m)

    def start_writes(j):
        h, btp, _ = q_of(j)
        for q in range(4):
            for dtl in range(2):
                for btl in range(2):
                    pltpu.async_copy(
                        outq_v.at[q].at[pl.ds(8 * dtl, 8),
                                        pl.ds(128 * btl, 128)],
                        out_hbm.at[h, q * 2 + dtl, btp * 2 + btl], s_o)

    def wait_writes(j):
        h, btp, _ = q_of(j)
        for q in range(4):
            for dtl in range(2):
                for btl in range(2):
                    pltpu.make_async_copy(
                        outq_v.at[q].at[pl.ds(8 * dtl, 8),
                                        pl.ds(128 * btl, 128)],
                        out_hbm.at[h, q * 2 + dtl, btp * 2 + btl], s_o
                    ).wait()

    start_idx(0, 0)
    start_idx(1, 1)
    wait_idx(0)
    fuse(0)
    start_gather(0)
    start_idx(2, 2)
    wait_idx(1)
    fuse(1)
    start_gather(1)
    wait_idx(2)
    fuse(2)
    start_gather(2)
    start_idx(0, 3)
    wait_gather(0)
    scatter_group(0)
    start_writes(0)

    @pl.loop(0, (PER_W - 1) // RN)
    def _grp(t):
        for rr in range(RN):
            j = 1 + t * RN + rr
            rj = (1 + rr) % RN
            r2 = rr % RN

            @pl.when(j <= PER_W - 3)
            def _():
                wait_idx(r2)
                fuse(r2)
                start_gather(r2)

            @pl.when(j <= PER_W - 4)
            def _():
                start_idx(rj, j + 3)

            wait_gather(rj)
            wait_writes(j - 1)
            scatter_group(rj)
            start_writes(j)

    wait_writes(PER_W - 1)


def kernel(x, weight):
    xf = x.T.reshape(B * H)
    wv = weight.reshape(500000, 128)
    out5 = _emb_lookup(xf, wv)
    return out5.transpose((2, 4, 0, 1, 3)).reshape(B, H, D)
